# baseline (speedup 1.0000x reference)
import functools

import jax
import jax.numpy as jnp
from jax import lax
from jax.experimental import pallas as pl
from jax.experimental.pallas import tpu as pltpu
from jax.experimental.pallas import tpu_sc as plsc

_B = 4
_S = 2048
_D = 2048
_L = 16
_NC = 2
_NS = 16
_NW = _NC * _NS
_P = _S // _NW
_K = 4
_NCHUNK = _P // _K
_NBUF = 3
_R = _B * _K

_mesh = plsc.VectorSubcoreMesh(core_axis_name="c", subcore_axis_name="s")


@functools.partial(
    pl.kernel,
    out_type=jax.ShapeDtypeStruct((_B, _S, _D), jnp.float32),
    mesh=_mesh,
    scratch_types=[
        pltpu.VMEM((_NCHUNK, _R), jnp.int32),
        pltpu.VMEM((_NBUF, _K, _D), jnp.float32),
        pltpu.VMEM((_NBUF, _R, _D), jnp.float32),
        pltpu.SemaphoreType.DMA,
        pltpu.SemaphoreType.DMA,
        pltpu.SemaphoreType.DMA,
        pltpu.SemaphoreType.DMA,
        pltpu.SemaphoreType.DMA,
        pltpu.SemaphoreType.DMA,
    ],
)
def _embed(ids_hbm, wte_hbm, wpe_hbm, out_hbm, idx_v, wpe_v, rows_v,
           sem_g0, sem_g1, sem_g2, sem_s0, sem_s1, sem_s2):
    sem_g = (sem_g0, sem_g1, sem_g2)
    sem_s = (sem_s0, sem_s1, sem_s2)
    wid = lax.axis_index("s") * _NC + lax.axis_index("c")
    s0 = wid * _P

    pltpu.sync_copy(ids_hbm.at[wid], idx_v)

    def issue(c, p):
        base = s0 + c * _K
        return [
            pltpu.async_copy(wpe_hbm.at[pl.ds(base, _K)], wpe_v.at[p],
                             sem_g[p]),
            pltpu.async_copy(wte_hbm.at[idx_v.at[c]], rows_v.at[p], sem_g[p]),
        ]

    _AHEAD = _NBUF - 1
    pending_g = {c: issue(c, c % _NBUF) for c in range(_AHEAD)}
    pending_s = {}
    for c in range(_NCHUNK):
        p = c % _NBUF
        if c + _AHEAD < _NCHUNK:
            q = (c + _AHEAD) % _NBUF
            if q in pending_s:
                for cp in pending_s.pop(q):
                    cp.wait()
            pending_g[c + _AHEAD] = issue(c + _AHEAD, q)
        for cp in pending_g.pop(c):
            cp.wait()

        base = s0 + c * _K
        stores = []
        for b in range(_B):
            @plsc.parallel_loop(0, _K * _D, _L, unroll=8)
            def _add(j, p=p, b=b):
                i = j // _D
                jj = lax.rem(j, _D)
                plsc.addupdate(rows_v.at[p, b * _K + i, pl.ds(jj, _L)],
                               wpe_v[p, i, pl.ds(jj, _L)])
            stores.append(
                pltpu.async_copy(rows_v.at[p, pl.ds(b * _K, _K)],
                                 out_hbm.at[b, pl.ds(base, _K)], sem_s[p]))
        pending_s[p] = stores
    for cps in pending_s.values():
        for cp in cps:
            cp.wait()


def kernel(input_ids, wte, wpe):
    ids_t = jnp.transpose(
        input_ids.reshape(_B, _NW, _NCHUNK, _K), (1, 2, 0, 3)
    ).reshape(_NW, _NCHUNK, _R)
    return _embed(ids_t, wte, wpe)


# speedup vs baseline: 1.0237x; 1.0237x over previous
import functools

import jax
import jax.numpy as jnp
from jax import lax
from jax.experimental import pallas as pl
from jax.experimental.pallas import tpu as pltpu
from jax.experimental.pallas import tpu_sc as plsc

_B = 4
_S = 2048
_D = 2048
_L = 16
_NC = 2
_NS = 16
_NW = _NC * _NS
_P = _S // _NW
_K = 4
_NCHUNK = _P // _K
_NBUF = 3
_R = _B * _K

_mesh = plsc.VectorSubcoreMesh(core_axis_name="c", subcore_axis_name="s")


@functools.partial(
    pl.kernel,
    out_type=jax.ShapeDtypeStruct((_B, _S, _D), jnp.float32),
    mesh=_mesh,
    scratch_types=[
        pltpu.VMEM((_NCHUNK, _R), jnp.int32),
        pltpu.VMEM((_NBUF, _K, _D), jnp.float32),
        pltpu.VMEM((_NBUF, _R, _D), jnp.float32),
        pltpu.SemaphoreType.DMA,
        pltpu.SemaphoreType.DMA,
        pltpu.SemaphoreType.DMA,
        pltpu.SemaphoreType.DMA,
        pltpu.SemaphoreType.DMA,
        pltpu.SemaphoreType.DMA,
    ],
)
def _embed(ids_hbm, wte_hbm, wpe_hbm, out_hbm, idx_v, wpe_v, rows_v,
           sem_g0, sem_g1, sem_g2, sem_s0, sem_s1, sem_s2):
    sem_g = (sem_g0, sem_g1, sem_g2)
    sem_s = (sem_s0, sem_s1, sem_s2)
    wid = lax.axis_index("s") * _NC + lax.axis_index("c")
    s0 = wid * _P

    pltpu.sync_copy(ids_hbm.at[wid], idx_v)

    def issue(c, p):
        base = s0 + c * _K
        return [
            pltpu.async_copy(wpe_hbm.at[pl.ds(base, _K)], wpe_v.at[p],
                             sem_g[p]),
            pltpu.async_copy(wte_hbm.at[idx_v.at[c]], rows_v.at[p], sem_g[p]),
        ]

    _AHEAD = _NBUF - 1
    pending_g = {c: issue(c, c % _NBUF) for c in range(_AHEAD)}
    pending_s = {}
    for c in range(_NCHUNK):
        p = c % _NBUF
        if c + _AHEAD < _NCHUNK:
            q = (c + _AHEAD) % _NBUF
            if q in pending_s:
                for cp in pending_s.pop(q):
                    cp.wait()
            pending_g[c + _AHEAD] = issue(c + _AHEAD, q)
        for cp in pending_g.pop(c):
            cp.wait()

        @plsc.parallel_loop(0, _K * _D, _L, unroll=8)
        def _add(j, p=p):
            i = j // _D
            jj = lax.rem(j, _D)
            v = wpe_v[p, i, pl.ds(jj, _L)]
            for b in range(_B):
                plsc.addupdate(rows_v.at[p, b * _K + i, pl.ds(jj, _L)], v)

        base = s0 + c * _K
        pending_s[p] = [
            pltpu.async_copy(rows_v.at[p, pl.ds(b * _K, _K)],
                             out_hbm.at[b, pl.ds(base, _K)], sem_s[p])
            for b in range(_B)
        ]
    for cps in pending_s.values():
        for cp in cps:
            cp.wait()


def kernel(input_ids, wte, wpe):
    ids_t = jnp.transpose(
        input_ids.reshape(_B, _NW, _NCHUNK, _K), (1, 2, 0, 3)
    ).reshape(_NW, _NCHUNK, _R)
    return _embed(ids_t, wte, wpe)


# speedup vs baseline: 1.0342x; 1.0103x over previous
import functools

import jax
import jax.numpy as jnp
from jax import lax
from jax.experimental import pallas as pl
from jax.experimental.pallas import tpu as pltpu
from jax.experimental.pallas import tpu_sc as plsc

_B = 4
_S = 2048
_D = 2048
_L = 16
_NC = 2
_NS = 16
_NW = _NC * _NS
_P = _S // _NW
_K = 4
_NCHUNK = _P // _K
_NBUF = 3
_R = _B * _K

_mesh = plsc.VectorSubcoreMesh(core_axis_name="c", subcore_axis_name="s")


@functools.partial(
    pl.kernel,
    out_type=jax.ShapeDtypeStruct((_B, _S, _D), jnp.float32),
    mesh=_mesh,
    scratch_types=[
        pltpu.VMEM((_NCHUNK, _R), jnp.int32),
        pltpu.VMEM((_NBUF, _K, _D), jnp.float32),
        pltpu.VMEM((_NBUF, _R, _D), jnp.float32),
        pltpu.SemaphoreType.DMA,
        pltpu.SemaphoreType.DMA,
        pltpu.SemaphoreType.DMA,
        pltpu.SemaphoreType.DMA,
        pltpu.SemaphoreType.DMA,
        pltpu.SemaphoreType.DMA,
    ],
)
def _embed(ids_hbm, wte_hbm, wpe_hbm, out_hbm, idx_v, wpe_v, rows_v,
           sem_g0, sem_g1, sem_g2, sem_s0, sem_s1, sem_s2):
    sem_g = (sem_g0, sem_g1, sem_g2)
    sem_s = (sem_s0, sem_s1, sem_s2)
    wid = lax.axis_index("s") * _NC + lax.axis_index("c")
    s0 = wid * _P

    def issue_wpe(c, p):
        base = s0 + c * _K
        return pltpu.async_copy(wpe_hbm.at[pl.ds(base, _K)], wpe_v.at[p],
                                sem_g[p])

    def issue(c, p):
        return [
            pltpu.async_copy(wte_hbm.at[idx_v.at[c]], rows_v.at[p], sem_g[p]),
            issue_wpe(c, p),
        ]

    _AHEAD = _NBUF - 1
    prolog_wpe = {c: issue_wpe(c, c % _NBUF) for c in range(_AHEAD)}
    pltpu.sync_copy(ids_hbm.at[wid], idx_v)
    pending_g = {
        c: [pltpu.async_copy(wte_hbm.at[idx_v.at[c]], rows_v.at[c % _NBUF],
                             sem_g[c % _NBUF]),
            prolog_wpe[c]]
        for c in range(_AHEAD)
    }
    pending_s = {}
    for c in range(_NCHUNK):
        p = c % _NBUF
        if c + _AHEAD < _NCHUNK:
            q = (c + _AHEAD) % _NBUF
            if q in pending_s:
                for cp in pending_s.pop(q):
                    cp.wait()
            pending_g[c + _AHEAD] = issue(c + _AHEAD, q)
        for cp in pending_g.pop(c):
            cp.wait()

        @plsc.parallel_loop(0, _K * _D, _L, unroll=4)
        def _add(j, p=p):
            i = j // _D
            jj = lax.rem(j, _D)
            v = wpe_v[p, i, pl.ds(jj, _L)]
            for b in range(_B):
                plsc.addupdate(rows_v.at[p, b * _K + i, pl.ds(jj, _L)], v)

        base = s0 + c * _K
        pending_s[p] = [
            pltpu.async_copy(rows_v.at[p, pl.ds(b * _K, _K)],
                             out_hbm.at[b, pl.ds(base, _K)], sem_s[p])
            for b in range(_B)
        ]
    for cps in pending_s.values():
        for cp in cps:
            cp.wait()


def kernel(input_ids, wte, wpe):
    ids_t = jnp.transpose(
        input_ids.reshape(_B, _NW, _NCHUNK, _K), (1, 2, 0, 3)
    ).reshape(_NW, _NCHUNK, _R)
    return _embed(ids_t, wte, wpe)


# speedup vs baseline: 1.0344x; 1.0002x over previous
import functools

import jax
import jax.numpy as jnp
from jax import lax
from jax.experimental import pallas as pl
from jax.experimental.pallas import tpu as pltpu
from jax.experimental.pallas import tpu_sc as plsc

_B = 4
_S = 2048
_D = 2048
_L = 16
_NC = 2
_NS = 16
_NW = _NC * _NS
_P = _S // _NW
_K = 4
_NCHUNK = _P // _K
_NBUF = 3
_R = _B * _K

_mesh = plsc.VectorSubcoreMesh(core_axis_name="c", subcore_axis_name="s")


@functools.partial(
    pl.kernel,
    out_type=jax.ShapeDtypeStruct((_B, _S, _D), jnp.float32),
    mesh=_mesh,
    scratch_types=[
        pltpu.VMEM((_NCHUNK, _R), jnp.int32),
        pltpu.VMEM((_NBUF, _K, _D), jnp.float32),
        pltpu.VMEM((_NBUF, _R, _D), jnp.float32),
        pltpu.SemaphoreType.DMA,
        pltpu.SemaphoreType.DMA,
        pltpu.SemaphoreType.DMA,
        pltpu.SemaphoreType.DMA,
        pltpu.SemaphoreType.DMA,
        pltpu.SemaphoreType.DMA,
    ],
)
def _embed(ids_hbm, wte_hbm, wpe_hbm, out_hbm, idx_v, wpe_v, rows_v,
           sem_g0, sem_g1, sem_g2, sem_s0, sem_s1, sem_s2):
    sem_g = (sem_g0, sem_g1, sem_g2)
    sem_s = (sem_s0, sem_s1, sem_s2)
    wid = lax.axis_index("s") * _NC + lax.axis_index("c")
    s0 = wid * _P

    def issue_wpe(c, p):
        base = s0 + c * _K
        return pltpu.async_copy(wpe_hbm.at[pl.ds(base, _K)], wpe_v.at[p],
                                sem_g[p])

    def issue(c, p):
        return [
            pltpu.async_copy(wte_hbm.at[idx_v.at[c, pl.ds(0, _R // 2)]],
                             rows_v.at[p, pl.ds(0, _R // 2)], sem_g[p]),
            pltpu.async_copy(wte_hbm.at[idx_v.at[c, pl.ds(_R // 2, _R // 2)]],
                             rows_v.at[p, pl.ds(_R // 2, _R // 2)], sem_g[p]),
            issue_wpe(c, p),
        ]

    _AHEAD = _NBUF - 1
    prolog_wpe = {c: issue_wpe(c, c % _NBUF) for c in range(_AHEAD)}
    pltpu.sync_copy(ids_hbm.at[wid], idx_v)
    pending_g = {
        c: [pltpu.async_copy(wte_hbm.at[idx_v.at[c, pl.ds(0, _R // 2)]],
                             rows_v.at[c % _NBUF, pl.ds(0, _R // 2)],
                             sem_g[c % _NBUF]),
            pltpu.async_copy(wte_hbm.at[idx_v.at[c, pl.ds(_R // 2, _R // 2)]],
                             rows_v.at[c % _NBUF, pl.ds(_R // 2, _R // 2)],
                             sem_g[c % _NBUF]),
            prolog_wpe[c]]
        for c in range(_AHEAD)
    }
    pending_s = {}
    for c in range(_NCHUNK):
        p = c % _NBUF
        if c + _AHEAD < _NCHUNK:
            q = (c + _AHEAD) % _NBUF
            if q in pending_s:
                for cp in pending_s.pop(q):
                    cp.wait()
            pending_g[c + _AHEAD] = issue(c + _AHEAD, q)
        for cp in pending_g.pop(c):
            cp.wait()

        @plsc.parallel_loop(0, _K * _D, _L, unroll=4)
        def _add(j, p=p):
            i = j // _D
            jj = lax.rem(j, _D)
            v = wpe_v[p, i, pl.ds(jj, _L)]
            for b in range(_B):
                plsc.addupdate(rows_v.at[p, b * _K + i, pl.ds(jj, _L)], v)

        base = s0 + c * _K
        pending_s[p] = [
            pltpu.async_copy(rows_v.at[p, pl.ds(b * _K, _K)],
                             out_hbm.at[b, pl.ds(base, _K)], sem_s[p])
            for b in range(_B)
        ]
    for cps in pending_s.values():
        for cp in cps:
            cp.wait()


def kernel(input_ids, wte, wpe):
    ids_t = jnp.transpose(
        input_ids.reshape(_B, _NW, _NCHUNK, _K), (1, 2, 0, 3)
    ).reshape(_NW, _NCHUNK, _R)
    return _embed(ids_t, wte, wpe)
